# baseline (speedup 1.0000x reference)
import functools

import numpy as np
import jax
import jax.numpy as jnp
from jax import lax
from jax.experimental import pallas as pl
from jax.experimental.pallas import tpu as pltpu
from jax.experimental.pallas import tpu_sc as plsc

N = 10000
N_PAD = 10112
E_SP = 160000
E_SEQ = 20000
EP_SP = 163840
EP_SEQ = 20480
C_S = 128
C_Z = 64
C_H = 16
NW = 32
CHUNK = 128
RPT = N_PAD // 16
F32 = jnp.float32
PH = lax.Precision.HIGHEST


def _dot(a, b):
    return jnp.dot(a, b, precision=PH, preferred_element_type=F32)


def _ln(x, g, b):
    mu = jnp.mean(x, axis=-1, keepdims=True)
    var = jnp.mean((x - mu) * (x - mu), axis=-1, keepdims=True)
    return (x - mu) * lax.rsqrt(var + 1e-5) * g + b


def _consts():
    c = {}
    a16 = np.zeros((16, 128), np.float32)
    b16 = np.zeros((16, 128), np.float32)
    for t in range(16):
        a16[t, t] = 1
        b16[t, 32 + t] = 1
    g12 = np.zeros((12, 128), np.float32)
    for t in range(12):
        g12[t, 16 + t] = 1
    m1 = np.zeros((1, 128), np.float32)
    m1[0, 28] = 1
    c["A16"], c["B16"], c["G12"], c["M1"] = a16, b16, g12, m1
    ps = np.zeros((128, 64), np.float32)
    pd = np.zeros((128, 64), np.float32)
    for i in range(4):
        for j in range(4):
            k = 4 * i + j
            for cc in range(3):
                ps[16 + 3 * i + cc, 4 * k + cc] = 1
                pd[16 + 3 * j + cc, 4 * k + cc] = 1
    ss = np.zeros((64, 16), np.float32)
    for k in range(16):
        for cc in range(4):
            ss[4 * k + cc, k] = 1
    qs = np.zeros((128, 16), np.float32)
    qd = np.zeros((128, 16), np.float32)
    for t in range(16):
        qs[t, t] = 1
        qd[32 + t, t] = 1
    ms = np.zeros((128, 1), np.float32)
    ms[28, 0] = 1
    c["PS"], c["PD"], c["SS"], c["QS"], c["QD"], c["MS"] = ps, pd, ss, qs, qd, ms
    for cp in range(3):
        ga = np.zeros((9, 12), np.float32)
        gb = np.zeros((12, 12), np.float32)
        for i in range(4):
            for cc in range(3):
                ga[3 * cc + cp, 3 * i + cc] = 1
                gb[3 * i + cp, 3 * i + cc] = 1
        c[f"GA{cp}"], c[f"GB{cp}"] = ga, gb
    t3 = np.zeros((3, 12), np.float32)
    for i in range(4):
        for cc in range(3):
            t3[cc, 3 * i + cc] = 1
    c["T3"] = t3
    for j in range(3):
        ra = np.zeros((9, 9), np.float32)
        rb = np.zeros((9, 9), np.float32)
        ta = np.zeros((9, 3), np.float32)
        tb = np.zeros((3, 3), np.float32)
        for i in range(3):
            for k in range(3):
                ra[3 * i + j, 3 * i + k] = 1
                rb[3 * j + k, 3 * i + k] = 1
            ta[3 * i + j, i] = 1
            tb[j, i] = 1
        c[f"RA{j}"], c[f"RB{j}"], c[f"TA{j}"], c[f"TB{j}"] = ra, rb, ta, tb
    h16 = np.zeros((16, 128), np.float32)
    for t in range(16):
        h16[t, t] = 1
    cm = np.zeros((1, 128), np.float32)
    cm[0, 16] = 1
    cn = np.zeros((1, 128), np.float32)
    cn[0, 17] = 1
    c["H16"], c["CM"], c["CN"] = h16, cm, cn
    sh16 = np.zeros((128, 16), np.float32)
    for t in range(16):
        sh16[t, t] = 1
    scm = np.zeros((128, 1), np.float32)
    scm[16, 0] = 1
    scn = np.zeros((128, 1), np.float32)
    scn[17, 0] = 1
    c["SH16"], c["SCM"], c["SCN"] = sh16, scm, scn
    return {k: jnp.asarray(v) for k, v in c.items()}


def _run_rows(body, row_args, w_args, out_trailing, block, rows):
    nb = rows // block
    in_specs = (
        [pl.BlockSpec((block, a.shape[1]), lambda i: (i, 0)) for a in row_args]
        + [pl.BlockSpec(w.shape, lambda i: (0, 0)) for w in w_args]
    )
    out_specs = [pl.BlockSpec((block, t), lambda i: (i, 0)) for t in out_trailing]
    out_shape = [jax.ShapeDtypeStruct((rows, t), F32) for t in out_trailing]
    f = pl.pallas_call(body, grid=(nb,), in_specs=in_specs,
                       out_specs=out_specs, out_shape=out_shape)
    return f(*row_args, *w_args)


def _node_pre(s, rot9, trans, maskc, wpts, w1a, w1b, c):
    def body(s_ref, rot_ref, tr_ref, mk_ref, wp, wa, wb,
             ga0, ga1, ga2, gb0, gb1, gb2, t3, a16, b16, g12, m1,
             tab_ref):
        s_ = s_ref[...]
        pts = _dot(s_, wp[...])
        rot_ = rot_ref[...]
        g = _dot(tr_ref[...], t3[...])
        for ga, gb in ((ga0, gb0), (ga1, gb1), (ga2, gb2)):
            g = g + _dot(rot_, ga[...]) * _dot(pts, gb[...])
        a = _dot(s_, wa[...])
        b = _dot(s_, wb[...])
        mk = mk_ref[...]
        tab_ref[...] = (_dot(a, a16[...]) + _dot(b, b16[...])
                        + _dot(g, g12[...]) + _dot(mk, m1[...]))

    return _run_rows(
        body, [s, rot9, trans, maskc],
        [wpts, w1a, w1b, c["GA0"], c["GA1"], c["GA2"], c["GB0"], c["GB1"],
         c["GB2"], c["T3"], c["A16"], c["B16"], c["G12"], c["M1"]],
        [128], 1000, N)[0]


def _edge_ipmp(sa, sb, z, w1c, w1d, bm1, c):
    def body(sa_ref, sb_ref, z_ref, ps, pd, ss, qs, qd, msk, wz, wd, b1,
             h16, cm, cn, m_ref):
        sa_ = sa_ref[...]
        sb_ = sb_ref[...]
        df = _dot(sa_, ps[...]) - _dot(sb_, pd[...])
        d2 = _dot(df * df, ss[...])
        dist = jnp.sqrt(d2 + 1e-8)
        h = (_dot(sa_, qs[...]) + _dot(sb_, qd[...]) + _dot(z_ref[...], wz[...])
             + _dot(dist, wd[...]) + b1[...])
        h = jnp.maximum(h, 0.0)
        mk = _dot(sa_, msk[...])
        one = mk * 0.0 + 1.0
        m_ref[...] = (_dot(h * mk, h16[...]) + _dot(mk, cm[...])
                      + _dot(one, cn[...]))

    ep = sa.shape[0]
    return _run_rows(
        body, [sa, sb, z],
        [c["PS"], c["PD"], c["SS"], c["QS"], c["QD"], c["MS"], w1c, w1d, bm1,
         c["H16"], c["CM"], c["CN"]],
        [128], 512, ep)[0]


def _node_mid(a0, a1, s0, maskc, rot9, trans,
              wm2, bm2, wout, bout, lg, lb, wpts, w1a, w1b, c):
    def body(a0_ref, a1_ref, s_ref, mk_ref, rot_ref, tr_ref,
             w2, b2, wo, bo, g_ln, b_ln, wp, wa, wb, sh16, scm, scn,
             ga0, ga1, ga2, gb0, gb1, gb2, t3, a16, b16, g12, m1,
             s1_ref, tab_ref):
        agg32 = a0_ref[...] + a1_ref[...]
        hsum = _dot(agg32, sh16[...])
        msum = _dot(agg32, scm[...])
        deg = _dot(agg32, scn[...])
        agg = (_dot(hsum, w2[...]) + _dot(msum, b2[...]))
        agg = agg / jnp.maximum(deg, 1.0)
        upd = _dot(agg, wo[...]) + bo[...]
        mk = mk_ref[...]
        s1 = _ln(s_ref[...] + upd * mk, g_ln[...], b_ln[...])
        pts = _dot(s1, wp[...])
        rot_ = rot_ref[...]
        g = _dot(tr_ref[...], t3[...])
        for ga, gb in ((ga0, gb0), (ga1, gb1), (ga2, gb2)):
            g = g + _dot(rot_, ga[...]) * _dot(pts, gb[...])
        a = _dot(s1, wa[...])
        b = _dot(s1, wb[...])
        s1_ref[...] = s1
        tab_ref[...] = (_dot(a, a16[...]) + _dot(b, b16[...])
                        + _dot(g, g12[...]) + _dot(mk, m1[...]))

    return _run_rows(
        body, [a0, a1, s0, maskc, rot9, trans],
        [wm2, bm2, wout, bout, lg, lb, wpts, w1a, w1b, c["SH16"], c["SCM"],
         c["SCN"], c["GA0"], c["GA1"], c["GA2"], c["GB0"], c["GB1"],
         c["GB2"], c["T3"], c["A16"], c["B16"], c["G12"], c["M1"]],
        [128, 128], 1000, N)


def _node_final(a0, a1, s1, maskc, noisec, rot9, trans, p, c):
    def body(a0_ref, a1_ref, s_ref, mk_ref, nz_ref, rot_ref,
             tr_ref, wm2_, bm2_, wo, bo, g2, b2l, w1, b1, w2, b2, w3, b3,
             tg, tb_ln, bbw, bbb, wne, bne, wns, bns, sh16, scm, scn,
             ra0, ra1, ra2, rb0, rb1, rb2, ta0, ta1, ta2, tb0, tb1, tb2,
             s3_ref, nr_ref, nt_ref, he_ref, hse_ref):
        agg32 = a0_ref[...] + a1_ref[...]
        hsum = _dot(agg32, sh16[...])
        msum = _dot(agg32, scm[...])
        deg = _dot(agg32, scn[...])
        agg = (_dot(hsum, wm2_[...]) + _dot(msum, bm2_[...]))
        agg = agg / jnp.maximum(deg, 1.0)
        upd = _dot(agg, wo[...]) + bo[...]
        mk = mk_ref[...]
        s2 = _ln(s_ref[...] + upd * mk, g2[...], b2l[...])
        t = jnp.maximum(_dot(s2, w1[...]) + b1[...], 0.0)
        t = jnp.maximum(_dot(t, w2[...]) + b2[...], 0.0)
        t = _dot(t, w3[...]) + b3[...]
        s3 = _ln(s2 + t, tg[...], tb_ln[...]) * mk
        nz = nz_ref[...]
        uv = (_dot(s3 * nz, bbw[...]) + bbb[...]) * nz
        v0 = uv[:, 0:1]
        v1 = uv[:, 1:2]
        v2 = uv[:, 2:3]
        inv = lax.rsqrt(1.0 + v0 * v0 + v1 * v1 + v2 * v2)
        w = inv
        x = v0 * inv
        y = v1 * inv
        zz = v2 * inv
        ru = jnp.concatenate([
            1 - 2 * (y * y + zz * zz), 2 * (x * y - zz * w), 2 * (x * zz + y * w),
            2 * (x * y + zz * w), 1 - 2 * (x * x + zz * zz), 2 * (y * zz - x * w),
            2 * (x * zz - y * w), 2 * (y * zz + x * w), 1 - 2 * (x * x + y * y),
        ], axis=1)
        rot_ = rot_ref[...]
        nr = (_dot(rot_, ra0[...]) * _dot(ru, rb0[...])
              + _dot(rot_, ra1[...]) * _dot(ru, rb1[...])
              + _dot(rot_, ra2[...]) * _dot(ru, rb2[...]))
        v3 = uv[:, 3:6]
        nt = tr_ref[...] + (_dot(rot_, ta0[...]) * _dot(v3, tb0[...])
                            + _dot(rot_, ta1[...]) * _dot(v3, tb1[...])
                            + _dot(rot_, ta2[...]) * _dot(v3, tb2[...]))
        s3_ref[...] = s3
        nr_ref[...] = nr
        nt_ref[...] = nt
        he_ref[...] = _dot(s3, wne[...]) + bne[...]
        hse_ref[...] = _dot(s3, wns[...]) + bns[...]

    return _run_rows(
        body, [a0, a1, s1, maskc, noisec, rot9, trans],
        [p["wm2"], p["bm2"], p["wo"], p["bo"], p["g2"], p["b2l"], p["w1"],
         p["b1"], p["w2"], p["b2"], p["w3"], p["b3"], p["tg"], p["tb"],
         p["bbw"], p["bbb"], p["wne"], p["bne"], p["wns"], p["bns"],
         c["SH16"], c["SCM"], c["SCN"],
         c["RA0"], c["RA1"], c["RA2"], c["RB0"], c["RB1"], c["RB2"],
         c["TA0"], c["TA1"], c["TA2"], c["TB0"], c["TB1"], c["TB2"]],
        [128, 9, 3, 64, 64], 1000, N)


def _edge_trans(hs, hd, z, w1s, w1d, w1z, b1, w2, b2, lg, lb):
    def body(hs_ref, hd_ref, z_ref, ws, wd, wz, b1_, w2_, b2_, g_, bl_,
             out_ref):
        z_ = z_ref[...]
        e = (_dot(hs_ref[...], ws[...]) + _dot(hd_ref[...], wd[...])
             + _dot(z_, wz[...]) + b1_[...])
        e = jnp.maximum(e, 0.0)
        e = _dot(e, w2_[...]) + b2_[...] + z_
        out_ref[...] = _ln(e, g_[...], bl_[...])

    ep = hs.shape[0]
    return _run_rows(body, [hs, hd, z], [w1s, w1d, w1z, b1, w2, b2, lg, lb],
                     [64], 512, ep)[0]


def _gather_pairs(tab, ia, ib):
    ep = ia.shape[0]
    epw = ep // NW
    nch = epw // CHUNK
    mesh = plsc.VectorSubcoreMesh(core_axis_name="c", subcore_axis_name="s")

    @functools.partial(
        pl.kernel,
        out_type=(jax.ShapeDtypeStruct((ep, 128), F32),
                  jax.ShapeDtypeStruct((ep, 128), F32)),
        mesh=mesh,
        scratch_types=[
            pltpu.VMEM((CHUNK,), jnp.int32), pltpu.VMEM((CHUNK,), jnp.int32),
            pltpu.VMEM((CHUNK, 128), F32), pltpu.VMEM((CHUNK, 128), F32),
            pltpu.SemaphoreType.DMA, pltpu.SemaphoreType.DMA,
        ])
    def gk(t_h, ia_h, ib_h, oa_h, ob_h, iva, ivb, ra, rb, sema, semb):
        wid = lax.axis_index("s") * 2 + lax.axis_index("c")

        def chunk(k, carry):
            base = wid * epw + k * CHUNK
            pltpu.sync_copy(ia_h.at[pl.ds(base, CHUNK)], iva)
            pltpu.sync_copy(ib_h.at[pl.ds(base, CHUNK)], ivb)
            ca = pltpu.async_copy(t_h.at[iva], ra, sema)
            cb = pltpu.async_copy(t_h.at[ivb], rb, semb)
            ca.wait()
            cb.wait()
            pltpu.sync_copy(ra, oa_h.at[pl.ds(base, CHUNK)])
            pltpu.sync_copy(rb, ob_h.at[pl.ds(base, CHUNK)])
            return carry

        lax.fori_loop(0, nch, chunk, 0)

    return gk(tab, ia, ib)


def _scatter_segsum(m, idx):
    ep = m.shape[0]
    w = m.shape[1]
    epw = ep // NW
    nch = epw // CHUNK
    z128 = jnp.zeros((RPT, w), F32)
    mesh = plsc.VectorSubcoreMesh(core_axis_name="c", subcore_axis_name="s")

    @functools.partial(
        pl.kernel,
        out_type=jax.ShapeDtypeStruct((2 * N_PAD, w), F32),
        mesh=mesh,
        scratch_types=[
            pltpu.VMEM((CHUNK, w), F32), pltpu.VMEM((CHUNK,), jnp.int32),
            pltpu.VMEM_SHARED((N_PAD, w), F32),
        ])
    def sk(m_h, idx_h, z128_h, oa_h, mbuf, iv, agg_sh):
        cid = lax.axis_index("c")
        sid = lax.axis_index("s")
        wid = sid * 2 + cid
        pltpu.sync_copy(z128_h, agg_sh.at[pl.ds(sid * RPT, RPT)])
        plsc.subcore_barrier()

        def chunk(k, carry):
            base = wid * epw + k * CHUNK
            pltpu.sync_copy(idx_h.at[pl.ds(base, CHUNK)], iv)
            pltpu.sync_copy(m_h.at[pl.ds(base, CHUNK)], mbuf)
            pltpu.sync_copy(mbuf, agg_sh.at[iv], add=True)
            return carry

        lax.fori_loop(0, nch, chunk, 0)
        plsc.subcore_barrier()
        pltpu.sync_copy(agg_sh.at[pl.ds(sid * RPT, RPT)],
                        oa_h.at[pl.ds(cid * N_PAD + sid * RPT, RPT)])

    return sk(m, idx, z128).reshape(2, N_PAD, w)


def _pad_rows(x, rows):
    return jnp.pad(x, ((0, rows - x.shape[0]), (0, 0)))


def _pad_idx(ix, ep, fill):
    return jnp.pad(ix, (0, ep - ix.shape[0]), constant_values=fill)


def _ipmp_tables_weights(p):
    w1 = p["W_m1"]
    return (p["W_pts"], w1[:C_S], w1[C_S:2 * C_S], w1[2 * C_S:2 * C_S + C_Z],
            w1[2 * C_S + C_Z:], p["b_m1"].reshape(1, -1), p["W_m2"],
            p["b_m2"].reshape(1, -1))


def kernel(node_features, rot, trans, edge_features, seq_edge_features,
           params, edge_index, seq_edge_index, x_mask, noising_mask,
           num_graphs):
    c = _consts()
    maskc = (~x_mask).astype(F32).reshape(N, 1)
    noisec = noising_mask.astype(F32).reshape(N, 1)
    rot9 = rot.reshape(N, 9)

    src_sp = _pad_idx(edge_index[0], EP_SP, 0)
    dst_sp = _pad_idx(edge_index[1], EP_SP, 0)
    dst_sp_sc = _pad_idx(edge_index[1], EP_SP, N)
    z_sp = _pad_rows(edge_features, EP_SP)
    src_sq = _pad_idx(seq_edge_index[0], EP_SEQ, 0)
    dst_sq = _pad_idx(seq_edge_index[1], EP_SEQ, 0)
    dst_sq_sc = _pad_idx(seq_edge_index[1], EP_SEQ, N)
    z_sq = _pad_rows(seq_edge_features, EP_SEQ)

    sp = params["spatial"]
    wpts, w1a, w1b, w1c, w1d, bm1, wm2, bm2 = _ipmp_tables_weights(sp)
    tab = _node_pre(node_features, rot9, trans, maskc, wpts, w1a, w1b, c)
    tab = _pad_rows(tab, N_PAD)
    sa, sb = _gather_pairs(tab, src_sp, dst_sp)
    m = _edge_ipmp(sa, sb, z_sp, w1c, w1d, bm1, c)
    agg_p = _scatter_segsum(m, dst_sp_sc)

    sq = params["seq"]
    wpts2, w2a, w2b, w2c, w2d, bm1q, wm2q, bm2q = _ipmp_tables_weights(sq)
    s1, tab2 = _node_mid(
        agg_p[0], agg_p[1], node_features, maskc, rot9,
        trans, wm2, bm2, sp["W_out"], sp["b_out"].reshape(1, -1),
        params["ln_s1_g"].reshape(1, -1), params["ln_s1_b"].reshape(1, -1),
        wpts2, w2a, w2b, c)
    tab2 = _pad_rows(tab2, N_PAD)
    sa2, sb2 = _gather_pairs(tab2, src_sq, dst_sq)
    m2 = _edge_ipmp(sa2, sb2, z_sq, w2c, w2d, bm1q, c)
    agg2_p = _scatter_segsum(m2, dst_sq_sc)

    tp = params["trans"]
    ep_, sp_ = params["edge"], params["seq_edge"]
    nf = {
        "wm2": wm2q, "bm2": bm2q,
        "wo": sq["W_out"], "bo": sq["b_out"].reshape(1, -1),
        "g2": params["ln_s2_g"].reshape(1, -1),
        "b2l": params["ln_s2_b"].reshape(1, -1),
        "w1": tp["W1"], "b1": tp["b1"].reshape(1, -1),
        "w2": tp["W2"], "b2": tp["b2"].reshape(1, -1),
        "w3": tp["W3"], "b3": tp["b3"].reshape(1, -1),
        "tg": tp["ln_g"].reshape(1, -1), "tb": tp["ln_b"].reshape(1, -1),
        "bbw": params["bb_W"], "bbb": params["bb_b"].reshape(1, -1),
        "wne": ep_["W_node"], "bne": ep_["b_node"].reshape(1, -1),
        "wns": sp_["W_node"], "bns": sp_["b_node"].reshape(1, -1),
    }
    s3, nr9, nt, he, hse = _node_final(
        agg2_p[0], agg2_p[1], s1, maskc, noisec, rot9, trans, nf, c)

    he_p = jnp.pad(he, ((0, N_PAD - N), (0, 128 - C_Z)))
    hse_p = jnp.pad(hse, ((0, N_PAD - N), (0, 128 - C_Z)))
    hs, hd = _gather_pairs(he_p, src_sp, dst_sp)
    pad64 = ((0, 64), (0, 0))
    ef = _edge_trans(hs, hd, z_sp, jnp.pad(ep_["W1"][:C_Z], pad64),
                     jnp.pad(ep_["W1"][C_Z:2 * C_Z], pad64),
                     ep_["W1"][2 * C_Z:], ep_["b1"].reshape(1, -1), ep_["W2"],
                     ep_["b2"].reshape(1, -1), ep_["ln_g"].reshape(1, -1),
                     ep_["ln_b"].reshape(1, -1))
    hs2, hd2 = _gather_pairs(hse_p, src_sq, dst_sq)
    sef = _edge_trans(hs2, hd2, z_sq, jnp.pad(sp_["W1"][:C_Z], pad64),
                      jnp.pad(sp_["W1"][C_Z:2 * C_Z], pad64),
                      sp_["W1"][2 * C_Z:], sp_["b1"].reshape(1, -1),
                      sp_["W2"], sp_["b2"].reshape(1, -1),
                      sp_["ln_g"].reshape(1, -1), sp_["ln_b"].reshape(1, -1))

    kl = jnp.zeros(8, F32)
    return (s3, nr9.reshape(N, 3, 3), nt, ef[:E_SP], sef[:E_SEQ], kl, kl)


# speedup vs baseline: 1.0486x; 1.0486x over previous
import functools

import numpy as np
import jax
import jax.numpy as jnp
from jax import lax
from jax.experimental import pallas as pl
from jax.experimental.pallas import tpu as pltpu
from jax.experimental.pallas import tpu_sc as plsc

N = 10000
N_PAD = 10112
E_SP = 160000
E_SEQ = 20000
EP_SP = 163840
EP_SEQ = 20480
C_S = 128
C_Z = 64
C_H = 16
NW = 32
CHUNK = 128
RPT = N_PAD // 16
F32 = jnp.float32
PH = lax.Precision.HIGHEST


def _dot(a, b):
    return jnp.dot(a, b, precision=PH, preferred_element_type=F32)


def _ln(x, g, b):
    mu = jnp.mean(x, axis=-1, keepdims=True)
    var = jnp.mean((x - mu) * (x - mu), axis=-1, keepdims=True)
    return (x - mu) * lax.rsqrt(var + 1e-5) * g + b


def _consts():
    c = {}
    a16 = np.zeros((16, 128), np.float32)
    b16 = np.zeros((16, 128), np.float32)
    for t in range(16):
        a16[t, t] = 1
        b16[t, 32 + t] = 1
    g12 = np.zeros((12, 128), np.float32)
    for t in range(12):
        g12[t, 16 + t] = 1
    m1 = np.zeros((1, 128), np.float32)
    m1[0, 28] = 1
    c["A16"], c["B16"], c["G12"], c["M1"] = a16, b16, g12, m1
    ps = np.zeros((128, 64), np.float32)
    pd = np.zeros((128, 64), np.float32)
    for i in range(4):
        for j in range(4):
            k = 4 * i + j
            for cc in range(3):
                ps[16 + 3 * i + cc, 4 * k + cc] = 1
                pd[16 + 3 * j + cc, 4 * k + cc] = 1
    ss = np.zeros((64, 16), np.float32)
    for k in range(16):
        for cc in range(4):
            ss[4 * k + cc, k] = 1
    qs = np.zeros((128, 16), np.float32)
    qd = np.zeros((128, 16), np.float32)
    for t in range(16):
        qs[t, t] = 1
        qd[32 + t, t] = 1
    ms = np.zeros((128, 1), np.float32)
    ms[28, 0] = 1
    c["PS"], c["PD"], c["SS"], c["QS"], c["QD"], c["MS"] = ps, pd, ss, qs, qd, ms
    for cp in range(3):
        ga = np.zeros((9, 12), np.float32)
        gb = np.zeros((12, 12), np.float32)
        for i in range(4):
            for cc in range(3):
                ga[3 * cc + cp, 3 * i + cc] = 1
                gb[3 * i + cp, 3 * i + cc] = 1
        c[f"GA{cp}"], c[f"GB{cp}"] = ga, gb
    t3 = np.zeros((3, 12), np.float32)
    for i in range(4):
        for cc in range(3):
            t3[cc, 3 * i + cc] = 1
    c["T3"] = t3
    for j in range(3):
        ra = np.zeros((9, 9), np.float32)
        rb = np.zeros((9, 9), np.float32)
        ta = np.zeros((9, 3), np.float32)
        tb = np.zeros((3, 3), np.float32)
        for i in range(3):
            for k in range(3):
                ra[3 * i + j, 3 * i + k] = 1
                rb[3 * j + k, 3 * i + k] = 1
            ta[3 * i + j, i] = 1
            tb[j, i] = 1
        c[f"RA{j}"], c[f"RB{j}"], c[f"TA{j}"], c[f"TB{j}"] = ra, rb, ta, tb
    h16 = np.zeros((16, 128), np.float32)
    for t in range(16):
        h16[t, t] = 1
    cm = np.zeros((1, 128), np.float32)
    cm[0, 16] = 1
    cn = np.zeros((1, 128), np.float32)
    cn[0, 17] = 1
    c["H16"], c["CM"], c["CN"] = h16, cm, cn
    sh16 = np.zeros((128, 16), np.float32)
    for t in range(16):
        sh16[t, t] = 1
    scm = np.zeros((128, 1), np.float32)
    scm[16, 0] = 1
    scn = np.zeros((128, 1), np.float32)
    scn[17, 0] = 1
    c["SH16"], c["SCM"], c["SCN"] = sh16, scm, scn
    return {k: jnp.asarray(v) for k, v in c.items()}


def _run_rows(body, row_args, w_args, out_trailing, block, rows):
    nb = rows // block
    in_specs = (
        [pl.BlockSpec((block, a.shape[1]), lambda i: (i, 0)) for a in row_args]
        + [pl.BlockSpec(w.shape, lambda i: (0, 0)) for w in w_args]
    )
    out_specs = [pl.BlockSpec((block, t), lambda i: (i, 0)) for t in out_trailing]
    out_shape = [jax.ShapeDtypeStruct((rows, t), F32) for t in out_trailing]
    f = pl.pallas_call(body, grid=(nb,), in_specs=in_specs,
                       out_specs=out_specs, out_shape=out_shape)
    return f(*row_args, *w_args)


def _node_pre(s, rot9, trans, maskc, wpts, w1a, w1b, c):
    def body(s_ref, rot_ref, tr_ref, mk_ref, wp, wa, wb,
             ga0, ga1, ga2, gb0, gb1, gb2, t3, a16, b16, g12, m1,
             tab_ref):
        s_ = s_ref[...]
        pts = _dot(s_, wp[...])
        rot_ = rot_ref[...]
        g = _dot(tr_ref[...], t3[...])
        for ga, gb in ((ga0, gb0), (ga1, gb1), (ga2, gb2)):
            g = g + _dot(rot_, ga[...]) * _dot(pts, gb[...])
        a = _dot(s_, wa[...])
        b = _dot(s_, wb[...])
        mk = mk_ref[...]
        tab_ref[...] = (_dot(a, a16[...]) + _dot(b, b16[...])
                        + _dot(g, g12[...]) + _dot(mk, m1[...]))

    return _run_rows(
        body, [s, rot9, trans, maskc],
        [wpts, w1a, w1b, c["GA0"], c["GA1"], c["GA2"], c["GB0"], c["GB1"],
         c["GB2"], c["T3"], c["A16"], c["B16"], c["G12"], c["M1"]],
        [128], 1000, N)[0]


def _edge_ipmp(sa, sb, z, w1c, w1d, bm1, c):
    def body(sa_ref, sb_ref, z_ref, ps, pd, ss, qs, qd, msk, wz, wd, b1,
             h16, cm, cn, m_ref):
        sa_ = sa_ref[...]
        sb_ = sb_ref[...]
        df = _dot(sa_, ps[...]) - _dot(sb_, pd[...])
        d2 = _dot(df * df, ss[...])
        dist = jnp.sqrt(d2 + 1e-8)
        h = (_dot(sa_, qs[...]) + _dot(sb_, qd[...]) + _dot(z_ref[...], wz[...])
             + _dot(dist, wd[...]) + b1[...])
        h = jnp.maximum(h, 0.0)
        mk = _dot(sa_, msk[...])
        one = mk * 0.0 + 1.0
        m_ref[...] = (_dot(h * mk, h16[...]) + _dot(mk, cm[...])
                      + _dot(one, cn[...]))

    ep = sa.shape[0]
    return _run_rows(
        body, [sa, sb, z],
        [c["PS"], c["PD"], c["SS"], c["QS"], c["QD"], c["MS"], w1c, w1d, bm1,
         c["H16"], c["CM"], c["CN"]],
        [128], 512, ep)[0]


def _node_mid(a0, a1, s0, maskc, rot9, trans,
              wm2, bm2, wout, bout, lg, lb, wpts, w1a, w1b, c):
    def body(a0_ref, a1_ref, s_ref, mk_ref, rot_ref, tr_ref,
             w2, b2, wo, bo, g_ln, b_ln, wp, wa, wb, sh16, scm, scn,
             ga0, ga1, ga2, gb0, gb1, gb2, t3, a16, b16, g12, m1,
             s1_ref, tab_ref):
        agg32 = a0_ref[...] + a1_ref[...]
        hsum = _dot(agg32, sh16[...])
        msum = _dot(agg32, scm[...])
        deg = _dot(agg32, scn[...])
        agg = (_dot(hsum, w2[...]) + _dot(msum, b2[...]))
        agg = agg / jnp.maximum(deg, 1.0)
        upd = _dot(agg, wo[...]) + bo[...]
        mk = mk_ref[...]
        s1 = _ln(s_ref[...] + upd * mk, g_ln[...], b_ln[...])
        pts = _dot(s1, wp[...])
        rot_ = rot_ref[...]
        g = _dot(tr_ref[...], t3[...])
        for ga, gb in ((ga0, gb0), (ga1, gb1), (ga2, gb2)):
            g = g + _dot(rot_, ga[...]) * _dot(pts, gb[...])
        a = _dot(s1, wa[...])
        b = _dot(s1, wb[...])
        s1_ref[...] = s1
        tab_ref[...] = (_dot(a, a16[...]) + _dot(b, b16[...])
                        + _dot(g, g12[...]) + _dot(mk, m1[...]))

    return _run_rows(
        body, [a0, a1, s0, maskc, rot9, trans],
        [wm2, bm2, wout, bout, lg, lb, wpts, w1a, w1b, c["SH16"], c["SCM"],
         c["SCN"], c["GA0"], c["GA1"], c["GA2"], c["GB0"], c["GB1"],
         c["GB2"], c["T3"], c["A16"], c["B16"], c["G12"], c["M1"]],
        [128, 128], 1000, N)


def _node_final(a0, a1, s1, maskc, noisec, rot9, trans, p, c):
    def body(a0_ref, a1_ref, s_ref, mk_ref, nz_ref, rot_ref,
             tr_ref, wm2_, bm2_, wo, bo, g2, b2l, w1, b1, w2, b2, w3, b3,
             tg, tb_ln, bbw, bbb, wne, bne, wns, bns, sh16, scm, scn,
             ra0, ra1, ra2, rb0, rb1, rb2, ta0, ta1, ta2, tb0, tb1, tb2,
             s3_ref, nr_ref, nt_ref, he_ref, hse_ref):
        agg32 = a0_ref[...] + a1_ref[...]
        hsum = _dot(agg32, sh16[...])
        msum = _dot(agg32, scm[...])
        deg = _dot(agg32, scn[...])
        agg = (_dot(hsum, wm2_[...]) + _dot(msum, bm2_[...]))
        agg = agg / jnp.maximum(deg, 1.0)
        upd = _dot(agg, wo[...]) + bo[...]
        mk = mk_ref[...]
        s2 = _ln(s_ref[...] + upd * mk, g2[...], b2l[...])
        t = jnp.maximum(_dot(s2, w1[...]) + b1[...], 0.0)
        t = jnp.maximum(_dot(t, w2[...]) + b2[...], 0.0)
        t = _dot(t, w3[...]) + b3[...]
        s3 = _ln(s2 + t, tg[...], tb_ln[...]) * mk
        nz = nz_ref[...]
        uv = (_dot(s3 * nz, bbw[...]) + bbb[...]) * nz
        v0 = uv[:, 0:1]
        v1 = uv[:, 1:2]
        v2 = uv[:, 2:3]
        inv = lax.rsqrt(1.0 + v0 * v0 + v1 * v1 + v2 * v2)
        w = inv
        x = v0 * inv
        y = v1 * inv
        zz = v2 * inv
        ru = jnp.concatenate([
            1 - 2 * (y * y + zz * zz), 2 * (x * y - zz * w), 2 * (x * zz + y * w),
            2 * (x * y + zz * w), 1 - 2 * (x * x + zz * zz), 2 * (y * zz - x * w),
            2 * (x * zz - y * w), 2 * (y * zz + x * w), 1 - 2 * (x * x + y * y),
        ], axis=1)
        rot_ = rot_ref[...]
        nr = (_dot(rot_, ra0[...]) * _dot(ru, rb0[...])
              + _dot(rot_, ra1[...]) * _dot(ru, rb1[...])
              + _dot(rot_, ra2[...]) * _dot(ru, rb2[...]))
        v3 = uv[:, 3:6]
        nt = tr_ref[...] + (_dot(rot_, ta0[...]) * _dot(v3, tb0[...])
                            + _dot(rot_, ta1[...]) * _dot(v3, tb1[...])
                            + _dot(rot_, ta2[...]) * _dot(v3, tb2[...]))
        s3_ref[...] = s3
        nr_ref[...] = nr
        nt_ref[...] = nt
        he_ref[...] = _dot(s3, wne[...]) + bne[...]
        hse_ref[...] = _dot(s3, wns[...]) + bns[...]

    return _run_rows(
        body, [a0, a1, s1, maskc, noisec, rot9, trans],
        [p["wm2"], p["bm2"], p["wo"], p["bo"], p["g2"], p["b2l"], p["w1"],
         p["b1"], p["w2"], p["b2"], p["w3"], p["b3"], p["tg"], p["tb"],
         p["bbw"], p["bbb"], p["wne"], p["bne"], p["wns"], p["bns"],
         c["SH16"], c["SCM"], c["SCN"],
         c["RA0"], c["RA1"], c["RA2"], c["RB0"], c["RB1"], c["RB2"],
         c["TA0"], c["TA1"], c["TA2"], c["TB0"], c["TB1"], c["TB2"]],
        [128, 9, 3, 64, 64], 1000, N)


def _edge_trans(hs, hd, z, w1s, w1d, w1z, b1, w2, b2, lg, lb):
    def body(hs_ref, hd_ref, z_ref, ws, wd, wz, b1_, w2_, b2_, g_, bl_,
             out_ref):
        z_ = z_ref[...]
        e = (_dot(hs_ref[...], ws[...]) + _dot(hd_ref[...], wd[...])
             + _dot(z_, wz[...]) + b1_[...])
        e = jnp.maximum(e, 0.0)
        e = _dot(e, w2_[...]) + b2_[...] + z_
        out_ref[...] = _ln(e, g_[...], bl_[...])

    ep = hs.shape[0]
    return _run_rows(body, [hs, hd, z], [w1s, w1d, w1z, b1, w2, b2, lg, lb],
                     [64], 512, ep)[0]


def _gather_pairs(tab, ia, ib):
    ep = ia.shape[0]
    epw = ep // NW
    nch = epw // CHUNK
    mesh = plsc.VectorSubcoreMesh(core_axis_name="c", subcore_axis_name="s")

    @functools.partial(
        pl.kernel,
        out_type=(jax.ShapeDtypeStruct((ep, 128), F32),
                  jax.ShapeDtypeStruct((ep, 128), F32)),
        mesh=mesh,
        scratch_types=[
            pltpu.VMEM((CHUNK,), jnp.int32), pltpu.VMEM((CHUNK,), jnp.int32),
            pltpu.VMEM((CHUNK,), jnp.int32), pltpu.VMEM((CHUNK,), jnp.int32),
            pltpu.VMEM((CHUNK, 128), F32), pltpu.VMEM((CHUNK, 128), F32),
            pltpu.VMEM((CHUNK, 128), F32), pltpu.VMEM((CHUNK, 128), F32),
            pltpu.SemaphoreType.DMA, pltpu.SemaphoreType.DMA,
            pltpu.SemaphoreType.DMA, pltpu.SemaphoreType.DMA,
            pltpu.SemaphoreType.DMA, pltpu.SemaphoreType.DMA,
        ])
    def gk(t_h, ia_h, ib_h, oa_h, ob_h, iva0, ivb0, iva1, ivb1,
           ra0, rb0, ra1, rb1, si0, si1, sg0, sg1, sw0, sw1):
        wid = lax.axis_index("s") * 2 + lax.axis_index("c")
        base0 = wid * epw
        iv = ((iva0, ivb0), (iva1, ivb1))
        rr = ((ra0, rb0), (ra1, rb1))
        si = (si0, si1)
        sg = (sg0, sg1)
        sw = (sw0, sw1)
        idx_c = [None, None]
        w_c = [None, None]
        idx_c[0] = (pltpu.async_copy(ia_h.at[pl.ds(base0, CHUNK)], iva0, si0),
                    pltpu.async_copy(ib_h.at[pl.ds(base0, CHUNK)], ivb0, si0))
        for k in range(nch):
            s = k & 1
            if k + 1 < nch:
                ns = 1 - s
                nb = base0 + (k + 1) * CHUNK
                idx_c[ns] = (
                    pltpu.async_copy(ia_h.at[pl.ds(nb, CHUNK)], iv[ns][0],
                                     si[ns]),
                    pltpu.async_copy(ib_h.at[pl.ds(nb, CHUNK)], iv[ns][1],
                                     si[ns]))
            if w_c[s] is not None:
                w_c[s][0].wait()
                w_c[s][1].wait()
            idx_c[s][0].wait()
            idx_c[s][1].wait()
            ga = pltpu.async_copy(t_h.at[iv[s][0]], rr[s][0], sg[s])
            gb = pltpu.async_copy(t_h.at[iv[s][1]], rr[s][1], sg[s])
            ga.wait()
            gb.wait()
            b = base0 + k * CHUNK
            w_c[s] = (pltpu.async_copy(rr[s][0], oa_h.at[pl.ds(b, CHUNK)],
                                       sw[s]),
                      pltpu.async_copy(rr[s][1], ob_h.at[pl.ds(b, CHUNK)],
                                       sw[s]))
        for s in (0, 1):
            if w_c[s] is not None:
                w_c[s][0].wait()
                w_c[s][1].wait()

    return gk(tab, ia, ib)


def _scatter_segsum(m, idx):
    ep = m.shape[0]
    w = m.shape[1]
    epw = ep // NW
    nch = epw // CHUNK
    z128 = jnp.zeros((RPT, w), F32)
    mesh = plsc.VectorSubcoreMesh(core_axis_name="c", subcore_axis_name="s")

    @functools.partial(
        pl.kernel,
        out_type=jax.ShapeDtypeStruct((2 * N_PAD, w), F32),
        mesh=mesh,
        scratch_types=[
            pltpu.VMEM((CHUNK, w), F32), pltpu.VMEM((CHUNK, w), F32),
            pltpu.VMEM((CHUNK,), jnp.int32), pltpu.VMEM((CHUNK,), jnp.int32),
            pltpu.VMEM_SHARED((N_PAD, w), F32),
            pltpu.SemaphoreType.DMA, pltpu.SemaphoreType.DMA,
            pltpu.SemaphoreType.DMA, pltpu.SemaphoreType.DMA,
        ])
    def sk(m_h, idx_h, z128_h, oa_h, mbuf0, mbuf1, iv0, iv1, agg_sh,
           sl0, sl1, sa0, sa1):
        cid = lax.axis_index("c")
        sid = lax.axis_index("s")
        wid = sid * 2 + cid
        base0 = wid * epw
        mb = (mbuf0, mbuf1)
        iv = (iv0, iv1)
        sl = (sl0, sl1)
        sa = (sa0, sa1)
        pltpu.sync_copy(z128_h, agg_sh.at[pl.ds(sid * RPT, RPT)])
        plsc.subcore_barrier()
        l_c = [None, None]
        a_c = [None, None]
        l_c[0] = (pltpu.async_copy(idx_h.at[pl.ds(base0, CHUNK)], iv0, sl0),
                  pltpu.async_copy(m_h.at[pl.ds(base0, CHUNK)], mbuf0, sl0))
        for k in range(nch):
            s = k & 1
            if k + 1 < nch:
                ns = 1 - s
                if a_c[ns] is not None:
                    a_c[ns].wait()
                nb = base0 + (k + 1) * CHUNK
                l_c[ns] = (
                    pltpu.async_copy(idx_h.at[pl.ds(nb, CHUNK)], iv[ns],
                                     sl[ns]),
                    pltpu.async_copy(m_h.at[pl.ds(nb, CHUNK)], mb[ns],
                                     sl[ns]))
            l_c[s][0].wait()
            l_c[s][1].wait()
            a_c[s] = pltpu.async_copy(mb[s], agg_sh.at[iv[s]], sa[s],
                                      add=True)
        for s in (0, 1):
            if a_c[s] is not None:
                a_c[s].wait()
        plsc.subcore_barrier()
        pltpu.sync_copy(agg_sh.at[pl.ds(sid * RPT, RPT)],
                        oa_h.at[pl.ds(cid * N_PAD + sid * RPT, RPT)])

    return sk(m, idx, z128).reshape(2, N_PAD, w)


def _pad_rows(x, rows):
    return jnp.pad(x, ((0, rows - x.shape[0]), (0, 0)))


def _pad_idx(ix, ep, fill):
    return jnp.pad(ix, (0, ep - ix.shape[0]), constant_values=fill)


def _ipmp_tables_weights(p):
    w1 = p["W_m1"]
    return (p["W_pts"], w1[:C_S], w1[C_S:2 * C_S], w1[2 * C_S:2 * C_S + C_Z],
            w1[2 * C_S + C_Z:], p["b_m1"].reshape(1, -1), p["W_m2"],
            p["b_m2"].reshape(1, -1))


def kernel(node_features, rot, trans, edge_features, seq_edge_features,
           params, edge_index, seq_edge_index, x_mask, noising_mask,
           num_graphs):
    c = _consts()
    maskc = (~x_mask).astype(F32).reshape(N, 1)
    noisec = noising_mask.astype(F32).reshape(N, 1)
    rot9 = rot.reshape(N, 9)

    src_sp = _pad_idx(edge_index[0], EP_SP, 0)
    dst_sp = _pad_idx(edge_index[1], EP_SP, 0)
    dst_sp_sc = _pad_idx(edge_index[1], EP_SP, N)
    z_sp = _pad_rows(edge_features, EP_SP)
    src_sq = _pad_idx(seq_edge_index[0], EP_SEQ, 0)
    dst_sq = _pad_idx(seq_edge_index[1], EP_SEQ, 0)
    dst_sq_sc = _pad_idx(seq_edge_index[1], EP_SEQ, N)
    z_sq = _pad_rows(seq_edge_features, EP_SEQ)

    sp = params["spatial"]
    wpts, w1a, w1b, w1c, w1d, bm1, wm2, bm2 = _ipmp_tables_weights(sp)
    tab = _node_pre(node_features, rot9, trans, maskc, wpts, w1a, w1b, c)
    tab = _pad_rows(tab, N_PAD)
    sa, sb = _gather_pairs(tab, src_sp, dst_sp)
    m = _edge_ipmp(sa, sb, z_sp, w1c, w1d, bm1, c)
    agg_p = _scatter_segsum(m, dst_sp_sc)

    sq = params["seq"]
    wpts2, w2a, w2b, w2c, w2d, bm1q, wm2q, bm2q = _ipmp_tables_weights(sq)
    s1, tab2 = _node_mid(
        agg_p[0], agg_p[1], node_features, maskc, rot9,
        trans, wm2, bm2, sp["W_out"], sp["b_out"].reshape(1, -1),
        params["ln_s1_g"].reshape(1, -1), params["ln_s1_b"].reshape(1, -1),
        wpts2, w2a, w2b, c)
    tab2 = _pad_rows(tab2, N_PAD)
    sa2, sb2 = _gather_pairs(tab2, src_sq, dst_sq)
    m2 = _edge_ipmp(sa2, sb2, z_sq, w2c, w2d, bm1q, c)
    agg2_p = _scatter_segsum(m2, dst_sq_sc)

    tp = params["trans"]
    ep_, sp_ = params["edge"], params["seq_edge"]
    nf = {
        "wm2": wm2q, "bm2": bm2q,
        "wo": sq["W_out"], "bo": sq["b_out"].reshape(1, -1),
        "g2": params["ln_s2_g"].reshape(1, -1),
        "b2l": params["ln_s2_b"].reshape(1, -1),
        "w1": tp["W1"], "b1": tp["b1"].reshape(1, -1),
        "w2": tp["W2"], "b2": tp["b2"].reshape(1, -1),
        "w3": tp["W3"], "b3": tp["b3"].reshape(1, -1),
        "tg": tp["ln_g"].reshape(1, -1), "tb": tp["ln_b"].reshape(1, -1),
        "bbw": params["bb_W"], "bbb": params["bb_b"].reshape(1, -1),
        "wne": ep_["W_node"], "bne": ep_["b_node"].reshape(1, -1),
        "wns": sp_["W_node"], "bns": sp_["b_node"].reshape(1, -1),
    }
    s3, nr9, nt, he, hse = _node_final(
        agg2_p[0], agg2_p[1], s1, maskc, noisec, rot9, trans, nf, c)

    he_p = jnp.pad(he, ((0, N_PAD - N), (0, 128 - C_Z)))
    hse_p = jnp.pad(hse, ((0, N_PAD - N), (0, 128 - C_Z)))
    hs, hd = _gather_pairs(he_p, src_sp, dst_sp)
    pad64 = ((0, 64), (0, 0))
    ef = _edge_trans(hs, hd, z_sp, jnp.pad(ep_["W1"][:C_Z], pad64),
                     jnp.pad(ep_["W1"][C_Z:2 * C_Z], pad64),
                     ep_["W1"][2 * C_Z:], ep_["b1"].reshape(1, -1), ep_["W2"],
                     ep_["b2"].reshape(1, -1), ep_["ln_g"].reshape(1, -1),
                     ep_["ln_b"].reshape(1, -1))
    hs2, hd2 = _gather_pairs(hse_p, src_sq, dst_sq)
    sef = _edge_trans(hs2, hd2, z_sq, jnp.pad(sp_["W1"][:C_Z], pad64),
                      jnp.pad(sp_["W1"][C_Z:2 * C_Z], pad64),
                      sp_["W1"][2 * C_Z:], sp_["b1"].reshape(1, -1),
                      sp_["W2"], sp_["b2"].reshape(1, -1),
                      sp_["ln_g"].reshape(1, -1), sp_["ln_b"].reshape(1, -1))

    kl = jnp.zeros(8, F32)
    return (s3, nr9.reshape(N, 3, 3), nt, ef[:E_SP], sef[:E_SEQ], kl, kl)


# speedup vs baseline: 1.1949x; 1.1395x over previous
import functools

import numpy as np
import jax
import jax.numpy as jnp
from jax import lax
from jax.experimental import pallas as pl
from jax.experimental.pallas import tpu as pltpu
from jax.experimental.pallas import tpu_sc as plsc

N = 10000
N_PAD = 10112
E_SP = 160000
E_SEQ = 20000
EP_SP = 163840
EP_SEQ = 20480
C_S = 128
C_Z = 64
C_H = 16
NW = 32
CHUNK = 128
RPT = N_PAD // 16
F32 = jnp.float32
PH = lax.Precision.HIGHEST


def _dot(a, b):
    return jnp.dot(a, b, precision=PH, preferred_element_type=F32)


def _dotf(a, b):
    return jnp.dot(a, b, preferred_element_type=F32)


def _ln(x, g, b):
    mu = jnp.mean(x, axis=-1, keepdims=True)
    var = jnp.mean((x - mu) * (x - mu), axis=-1, keepdims=True)
    return (x - mu) * lax.rsqrt(var + 1e-5) * g + b


def _consts():
    c = {}
    a16 = np.zeros((16, 128), np.float32)
    b16 = np.zeros((16, 128), np.float32)
    for t in range(16):
        a16[t, t] = 1
        b16[t, 32 + t] = 1
    g12 = np.zeros((12, 128), np.float32)
    for t in range(12):
        g12[t, 16 + t] = 1
    m1 = np.zeros((1, 128), np.float32)
    m1[0, 28] = 1
    c["A16"], c["B16"], c["G12"], c["M1"] = a16, b16, g12, m1
    ps = np.zeros((128, 64), np.float32)
    pd = np.zeros((128, 64), np.float32)
    for i in range(4):
        for j in range(4):
            k = 4 * i + j
            for cc in range(3):
                ps[16 + 3 * i + cc, 4 * k + cc] = 1
                pd[16 + 3 * j + cc, 4 * k + cc] = 1
    ss = np.zeros((64, 16), np.float32)
    for k in range(16):
        for cc in range(4):
            ss[4 * k + cc, k] = 1
    qs = np.zeros((128, 16), np.float32)
    qd = np.zeros((128, 16), np.float32)
    for t in range(16):
        qs[t, t] = 1
        qd[32 + t, t] = 1
    ms = np.zeros((128, 1), np.float32)
    ms[28, 0] = 1
    c["PS"], c["PD"], c["SS"], c["QS"], c["QD"], c["MS"] = ps, pd, ss, qs, qd, ms
    for cp in range(3):
        ga = np.zeros((9, 12), np.float32)
        gb = np.zeros((12, 12), np.float32)
        for i in range(4):
            for cc in range(3):
                ga[3 * cc + cp, 3 * i + cc] = 1
                gb[3 * i + cp, 3 * i + cc] = 1
        c[f"GA{cp}"], c[f"GB{cp}"] = ga, gb
    t3 = np.zeros((3, 12), np.float32)
    for i in range(4):
        for cc in range(3):
            t3[cc, 3 * i + cc] = 1
    c["T3"] = t3
    for j in range(3):
        ra = np.zeros((9, 9), np.float32)
        rb = np.zeros((9, 9), np.float32)
        ta = np.zeros((9, 3), np.float32)
        tb = np.zeros((3, 3), np.float32)
        for i in range(3):
            for k in range(3):
                ra[3 * i + j, 3 * i + k] = 1
                rb[3 * j + k, 3 * i + k] = 1
            ta[3 * i + j, i] = 1
            tb[j, i] = 1
        c[f"RA{j}"], c[f"RB{j}"], c[f"TA{j}"], c[f"TB{j}"] = ra, rb, ta, tb
    h16 = np.zeros((16, 128), np.float32)
    for t in range(16):
        h16[t, t] = 1
    cm = np.zeros((1, 128), np.float32)
    cm[0, 16] = 1
    cn = np.zeros((1, 128), np.float32)
    cn[0, 17] = 1
    c["H16"], c["CM"], c["CN"] = h16, cm, cn
    sh16 = np.zeros((128, 16), np.float32)
    for t in range(16):
        sh16[t, t] = 1
    scm = np.zeros((128, 1), np.float32)
    scm[16, 0] = 1
    scn = np.zeros((128, 1), np.float32)
    scn[17, 0] = 1
    c["SH16"], c["SCM"], c["SCN"] = sh16, scm, scn
    return {k: jnp.asarray(v) for k, v in c.items()}


def _run_rows(body, row_args, w_args, out_trailing, block, rows):
    nb = rows // block
    in_specs = (
        [pl.BlockSpec((block, a.shape[1]), lambda i: (i, 0)) for a in row_args]
        + [pl.BlockSpec(w.shape, lambda i: (0, 0)) for w in w_args]
    )
    out_specs = [pl.BlockSpec((block, t), lambda i: (i, 0)) for t in out_trailing]
    out_shape = [jax.ShapeDtypeStruct((rows, t), F32) for t in out_trailing]
    f = pl.pallas_call(body, grid=(nb,), in_specs=in_specs,
                       out_specs=out_specs, out_shape=out_shape)
    return f(*row_args, *w_args)


def _node_pre(s, rot9, trans, maskc, wpts, w1a, w1b, c):
    def body(s_ref, rot_ref, tr_ref, mk_ref, wp, wa, wb,
             ga0, ga1, ga2, gb0, gb1, gb2, t3, a16, b16, g12, m1,
             tab_ref):
        s_ = s_ref[...]
        pts = _dot(s_, wp[...])
        rot_ = rot_ref[...]
        g = _dot(tr_ref[...], t3[...])
        for ga, gb in ((ga0, gb0), (ga1, gb1), (ga2, gb2)):
            g = g + _dot(rot_, ga[...]) * _dot(pts, gb[...])
        a = _dotf(s_, wa[...])
        b = _dotf(s_, wb[...])
        mk = mk_ref[...]
        tab_ref[...] = (_dot(a, a16[...]) + _dot(b, b16[...])
                        + _dot(g, g12[...]) + _dot(mk, m1[...]))

    return _run_rows(
        body, [s, rot9, trans, maskc],
        [wpts, w1a, w1b, c["GA0"], c["GA1"], c["GA2"], c["GB0"], c["GB1"],
         c["GB2"], c["T3"], c["A16"], c["B16"], c["G12"], c["M1"]],
        [128], 1000, N)[0]


def _edge_ipmp(sa, sb, z, w1c, w1d, bm1, c):
    def body(sa_ref, sb_ref, z_ref, ps, pd, ss, qs, qd, msk, wz, wd, b1,
             h16, cm, cn, m_ref):
        sa_ = sa_ref[...]
        sb_ = sb_ref[...]
        df = _dot(sa_, ps[...]) - _dot(sb_, pd[...])
        d2 = _dot(df * df, ss[...])
        dist = jnp.sqrt(d2 + 1e-8)
        h = (_dot(sa_, qs[...]) + _dot(sb_, qd[...]) + _dotf(z_ref[...], wz[...])
             + _dot(dist, wd[...]) + b1[...])
        h = jnp.maximum(h, 0.0)
        mk = _dot(sa_, msk[...])
        one = mk * 0.0 + 1.0
        m_ref[...] = (_dot(h * mk, h16[...]) + _dot(mk, cm[...])
                      + _dot(one, cn[...]))

    ep = sa.shape[0]
    return _run_rows(
        body, [sa, sb, z],
        [c["PS"], c["PD"], c["SS"], c["QS"], c["QD"], c["MS"], w1c, w1d, bm1,
         c["H16"], c["CM"], c["CN"]],
        [128], 512, ep)[0]


def _node_mid(a0, a1, s0, maskc, rot9, trans,
              wm2, bm2, wout, bout, lg, lb, wpts, w1a, w1b, c):
    def body(a0_ref, a1_ref, s_ref, mk_ref, rot_ref, tr_ref,
             w2, b2, wo, bo, g_ln, b_ln, wp, wa, wb, sh16, scm, scn,
             ga0, ga1, ga2, gb0, gb1, gb2, t3, a16, b16, g12, m1,
             s1_ref, tab_ref):
        agg32 = a0_ref[...] + a1_ref[...]
        hsum = _dot(agg32, sh16[...])
        msum = _dot(agg32, scm[...])
        deg = _dot(agg32, scn[...])
        agg = (_dotf(hsum, w2[...]) + _dot(msum, b2[...]))
        agg = agg / jnp.maximum(deg, 1.0)
        upd = _dotf(agg, wo[...]) + bo[...]
        mk = mk_ref[...]
        s1 = _ln(s_ref[...] + upd * mk, g_ln[...], b_ln[...])
        pts = _dot(s1, wp[...])
        rot_ = rot_ref[...]
        g = _dot(tr_ref[...], t3[...])
        for ga, gb in ((ga0, gb0), (ga1, gb1), (ga2, gb2)):
            g = g + _dot(rot_, ga[...]) * _dot(pts, gb[...])
        a = _dotf(s1, wa[...])
        b = _dotf(s1, wb[...])
        s1_ref[...] = s1
        tab_ref[...] = (_dot(a, a16[...]) + _dot(b, b16[...])
                        + _dot(g, g12[...]) + _dot(mk, m1[...]))

    return _run_rows(
        body, [a0, a1, s0, maskc, rot9, trans],
        [wm2, bm2, wout, bout, lg, lb, wpts, w1a, w1b, c["SH16"], c["SCM"],
         c["SCN"], c["GA0"], c["GA1"], c["GA2"], c["GB0"], c["GB1"],
         c["GB2"], c["T3"], c["A16"], c["B16"], c["G12"], c["M1"]],
        [128, 128], 1000, N)


def _node_final(a0, a1, s1, maskc, noisec, rot9, trans, p, c):
    def body(a0_ref, a1_ref, s_ref, mk_ref, nz_ref, rot_ref,
             tr_ref, wm2_, bm2_, wo, bo, g2, b2l, w1, b1, w2, b2, w3, b3,
             tg, tb_ln, bbw, bbb, wne, bne, wns, bns, sh16, scm, scn,
             ra0, ra1, ra2, rb0, rb1, rb2, ta0, ta1, ta2, tb0, tb1, tb2,
             s3_ref, nr_ref, nt_ref, he_ref, hse_ref):
        agg32 = a0_ref[...] + a1_ref[...]
        hsum = _dot(agg32, sh16[...])
        msum = _dot(agg32, scm[...])
        deg = _dot(agg32, scn[...])
        agg = (_dotf(hsum, wm2_[...]) + _dot(msum, bm2_[...]))
        agg = agg / jnp.maximum(deg, 1.0)
        upd = _dotf(agg, wo[...]) + bo[...]
        mk = mk_ref[...]
        s2 = _ln(s_ref[...] + upd * mk, g2[...], b2l[...])
        t = jnp.maximum(_dotf(s2, w1[...]) + b1[...], 0.0)
        t = jnp.maximum(_dotf(t, w2[...]) + b2[...], 0.0)
        t = _dotf(t, w3[...]) + b3[...]
        s3 = _ln(s2 + t, tg[...], tb_ln[...]) * mk
        nz = nz_ref[...]
        uv = (_dot(s3 * nz, bbw[...]) + bbb[...]) * nz
        v0 = uv[:, 0:1]
        v1 = uv[:, 1:2]
        v2 = uv[:, 2:3]
        inv = lax.rsqrt(1.0 + v0 * v0 + v1 * v1 + v2 * v2)
        w = inv
        x = v0 * inv
        y = v1 * inv
        zz = v2 * inv
        ru = jnp.concatenate([
            1 - 2 * (y * y + zz * zz), 2 * (x * y - zz * w), 2 * (x * zz + y * w),
            2 * (x * y + zz * w), 1 - 2 * (x * x + zz * zz), 2 * (y * zz - x * w),
            2 * (x * zz - y * w), 2 * (y * zz + x * w), 1 - 2 * (x * x + y * y),
        ], axis=1)
        rot_ = rot_ref[...]
        nr = (_dot(rot_, ra0[...]) * _dot(ru, rb0[...])
              + _dot(rot_, ra1[...]) * _dot(ru, rb1[...])
              + _dot(rot_, ra2[...]) * _dot(ru, rb2[...]))
        v3 = uv[:, 3:6]
        nt = tr_ref[...] + (_dot(rot_, ta0[...]) * _dot(v3, tb0[...])
                            + _dot(rot_, ta1[...]) * _dot(v3, tb1[...])
                            + _dot(rot_, ta2[...]) * _dot(v3, tb2[...]))
        s3_ref[...] = s3
        nr_ref[...] = nr
        nt_ref[...] = nt
        he_ref[...] = _dotf(s3, wne[...]) + bne[...]
        hse_ref[...] = _dotf(s3, wns[...]) + bns[...]

    return _run_rows(
        body, [a0, a1, s1, maskc, noisec, rot9, trans],
        [p["wm2"], p["bm2"], p["wo"], p["bo"], p["g2"], p["b2l"], p["w1"],
         p["b1"], p["w2"], p["b2"], p["w3"], p["b3"], p["tg"], p["tb"],
         p["bbw"], p["bbb"], p["wne"], p["bne"], p["wns"], p["bns"],
         c["SH16"], c["SCM"], c["SCN"],
         c["RA0"], c["RA1"], c["RA2"], c["RB0"], c["RB1"], c["RB2"],
         c["TA0"], c["TA1"], c["TA2"], c["TB0"], c["TB1"], c["TB2"]],
        [128, 9, 3, 64, 64], 1000, N)


def _edge_trans(hs, hd, z, w1s, w1d, w1z, b1, w2, b2, lg, lb):
    def body(hs_ref, hd_ref, z_ref, ws, wd, wz, b1_, w2_, b2_, g_, bl_,
             out_ref):
        z_ = z_ref[...]
        e = (_dotf(hs_ref[...], ws[...]) + _dotf(hd_ref[...], wd[...])
             + _dotf(z_, wz[...]) + b1_[...])
        e = jnp.maximum(e, 0.0)
        e = _dotf(e, w2_[...]) + b2_[...] + z_
        out_ref[...] = _ln(e, g_[...], bl_[...])

    ep = hs.shape[0]
    return _run_rows(body, [hs, hd, z], [w1s, w1d, w1z, b1, w2, b2, lg, lb],
                     [64], 512, ep)[0]


def _gather_pairs(tab, ia, ib):
    ep = ia.shape[0]
    epw = ep // NW
    nch = epw // CHUNK
    mesh = plsc.VectorSubcoreMesh(core_axis_name="c", subcore_axis_name="s")

    @functools.partial(
        pl.kernel,
        out_type=(jax.ShapeDtypeStruct((ep, 128), F32),
                  jax.ShapeDtypeStruct((ep, 128), F32)),
        mesh=mesh,
        scratch_types=[
            pltpu.VMEM((CHUNK,), jnp.int32), pltpu.VMEM((CHUNK,), jnp.int32),
            pltpu.VMEM((CHUNK,), jnp.int32), pltpu.VMEM((CHUNK,), jnp.int32),
            pltpu.VMEM((CHUNK, 128), F32), pltpu.VMEM((CHUNK, 128), F32),
            pltpu.VMEM((CHUNK, 128), F32), pltpu.VMEM((CHUNK, 128), F32),
            pltpu.SemaphoreType.DMA, pltpu.SemaphoreType.DMA,
            pltpu.SemaphoreType.DMA, pltpu.SemaphoreType.DMA,
            pltpu.SemaphoreType.DMA, pltpu.SemaphoreType.DMA,
        ])
    def gk(t_h, ia_h, ib_h, oa_h, ob_h, iva0, ivb0, iva1, ivb1,
           ra0, rb0, ra1, rb1, si0, si1, sg0, sg1, sw0, sw1):
        wid = lax.axis_index("s") * 2 + lax.axis_index("c")
        base0 = wid * epw
        iv = ((iva0, ivb0), (iva1, ivb1))
        rr = ((ra0, rb0), (ra1, rb1))
        si = (si0, si1)
        sg = (sg0, sg1)
        sw = (sw0, sw1)
        idx_c = [None, None]
        w_c = [None, None]
        idx_c[0] = (pltpu.async_copy(ia_h.at[pl.ds(base0, CHUNK)], iva0, si0),
                    pltpu.async_copy(ib_h.at[pl.ds(base0, CHUNK)], ivb0, si0))
        for k in range(nch):
            s = k & 1
            if k + 1 < nch:
                ns = 1 - s
                nb = base0 + (k + 1) * CHUNK
                idx_c[ns] = (
                    pltpu.async_copy(ia_h.at[pl.ds(nb, CHUNK)], iv[ns][0],
                                     si[ns]),
                    pltpu.async_copy(ib_h.at[pl.ds(nb, CHUNK)], iv[ns][1],
                                     si[ns]))
            if w_c[s] is not None:
                w_c[s][0].wait()
                w_c[s][1].wait()
            idx_c[s][0].wait()
            idx_c[s][1].wait()
            ga = pltpu.async_copy(t_h.at[iv[s][0]], rr[s][0], sg[s])
            gb = pltpu.async_copy(t_h.at[iv[s][1]], rr[s][1], sg[s])
            ga.wait()
            gb.wait()
            b = base0 + k * CHUNK
            w_c[s] = (pltpu.async_copy(rr[s][0], oa_h.at[pl.ds(b, CHUNK)],
                                       sw[s]),
                      pltpu.async_copy(rr[s][1], ob_h.at[pl.ds(b, CHUNK)],
                                       sw[s]))
        for s in (0, 1):
            if w_c[s] is not None:
                w_c[s][0].wait()
                w_c[s][1].wait()

    return gk(tab, ia, ib)


def _scatter_segsum(m, idx):
    ep = m.shape[0]
    w = m.shape[1]
    epw = ep // NW
    nch = epw // CHUNK
    z128 = jnp.zeros((RPT, w), F32)
    mesh = plsc.VectorSubcoreMesh(core_axis_name="c", subcore_axis_name="s")

    @functools.partial(
        pl.kernel,
        out_type=jax.ShapeDtypeStruct((2 * N_PAD, w), F32),
        mesh=mesh,
        scratch_types=[
            pltpu.VMEM((CHUNK, w), F32), pltpu.VMEM((CHUNK, w), F32),
            pltpu.VMEM((CHUNK,), jnp.int32), pltpu.VMEM((CHUNK,), jnp.int32),
            pltpu.VMEM_SHARED((N_PAD, w), F32),
            pltpu.SemaphoreType.DMA, pltpu.SemaphoreType.DMA,
            pltpu.SemaphoreType.DMA, pltpu.SemaphoreType.DMA,
        ])
    def sk(m_h, idx_h, z128_h, oa_h, mbuf0, mbuf1, iv0, iv1, agg_sh,
           sl0, sl1, sa0, sa1):
        cid = lax.axis_index("c")
        sid = lax.axis_index("s")
        wid = sid * 2 + cid
        base0 = wid * epw
        mb = (mbuf0, mbuf1)
        iv = (iv0, iv1)
        sl = (sl0, sl1)
        sa = (sa0, sa1)
        pltpu.sync_copy(z128_h, agg_sh.at[pl.ds(sid * RPT, RPT)])
        plsc.subcore_barrier()
        l_c = [None, None]
        a_c = [None, None]
        l_c[0] = (pltpu.async_copy(idx_h.at[pl.ds(base0, CHUNK)], iv0, sl0),
                  pltpu.async_copy(m_h.at[pl.ds(base0, CHUNK)], mbuf0, sl0))
        for k in range(nch):
            s = k & 1
            if k + 1 < nch:
                ns = 1 - s
                if a_c[ns] is not None:
                    a_c[ns].wait()
                nb = base0 + (k + 1) * CHUNK
                l_c[ns] = (
                    pltpu.async_copy(idx_h.at[pl.ds(nb, CHUNK)], iv[ns],
                                     sl[ns]),
                    pltpu.async_copy(m_h.at[pl.ds(nb, CHUNK)], mb[ns],
                                     sl[ns]))
            l_c[s][0].wait()
            l_c[s][1].wait()
            a_c[s] = pltpu.async_copy(mb[s], agg_sh.at[iv[s]], sa[s],
                                      add=True)
        for s in (0, 1):
            if a_c[s] is not None:
                a_c[s].wait()
        plsc.subcore_barrier()
        pltpu.sync_copy(agg_sh.at[pl.ds(sid * RPT, RPT)],
                        oa_h.at[pl.ds(cid * N_PAD + sid * RPT, RPT)])

    return sk(m, idx, z128).reshape(2, N_PAD, w)


def _pad_rows(x, rows):
    return jnp.pad(x, ((0, rows - x.shape[0]), (0, 0)))


def _pad_idx(ix, ep, fill):
    return jnp.pad(ix, (0, ep - ix.shape[0]), constant_values=fill)


def _ipmp_tables_weights(p):
    w1 = p["W_m1"]
    return (p["W_pts"], w1[:C_S], w1[C_S:2 * C_S], w1[2 * C_S:2 * C_S + C_Z],
            w1[2 * C_S + C_Z:], p["b_m1"].reshape(1, -1), p["W_m2"],
            p["b_m2"].reshape(1, -1))


def kernel(node_features, rot, trans, edge_features, seq_edge_features,
           params, edge_index, seq_edge_index, x_mask, noising_mask,
           num_graphs):
    c = _consts()
    maskc = (~x_mask).astype(F32).reshape(N, 1)
    noisec = noising_mask.astype(F32).reshape(N, 1)
    rot9 = rot.reshape(N, 9)

    src_sp = _pad_idx(edge_index[0], EP_SP, 0)
    dst_sp = _pad_idx(edge_index[1], EP_SP, 0)
    dst_sp_sc = _pad_idx(edge_index[1], EP_SP, N)
    z_sp = _pad_rows(edge_features, EP_SP)
    src_sq = _pad_idx(seq_edge_index[0], EP_SEQ, 0)
    dst_sq = _pad_idx(seq_edge_index[1], EP_SEQ, 0)
    dst_sq_sc = _pad_idx(seq_edge_index[1], EP_SEQ, N)
    z_sq = _pad_rows(seq_edge_features, EP_SEQ)

    sp = params["spatial"]
    wpts, w1a, w1b, w1c, w1d, bm1, wm2, bm2 = _ipmp_tables_weights(sp)
    tab = _node_pre(node_features, rot9, trans, maskc, wpts, w1a, w1b, c)
    tab = _pad_rows(tab, N_PAD)
    sa, sb = _gather_pairs(tab, src_sp, dst_sp)
    m = _edge_ipmp(sa, sb, z_sp, w1c, w1d, bm1, c)
    agg_p = _scatter_segsum(m, dst_sp_sc)

    sq = params["seq"]
    wpts2, w2a, w2b, w2c, w2d, bm1q, wm2q, bm2q = _ipmp_tables_weights(sq)
    s1, tab2 = _node_mid(
        agg_p[0], agg_p[1], node_features, maskc, rot9,
        trans, wm2, bm2, sp["W_out"], sp["b_out"].reshape(1, -1),
        params["ln_s1_g"].reshape(1, -1), params["ln_s1_b"].reshape(1, -1),
        wpts2, w2a, w2b, c)
    tab2 = _pad_rows(tab2, N_PAD)
    sa2, sb2 = _gather_pairs(tab2, src_sq, dst_sq)
    m2 = _edge_ipmp(sa2, sb2, z_sq, w2c, w2d, bm1q, c)
    agg2_p = _scatter_segsum(m2, dst_sq_sc)

    tp = params["trans"]
    ep_, sp_ = params["edge"], params["seq_edge"]
    nf = {
        "wm2": wm2q, "bm2": bm2q,
        "wo": sq["W_out"], "bo": sq["b_out"].reshape(1, -1),
        "g2": params["ln_s2_g"].reshape(1, -1),
        "b2l": params["ln_s2_b"].reshape(1, -1),
        "w1": tp["W1"], "b1": tp["b1"].reshape(1, -1),
        "w2": tp["W2"], "b2": tp["b2"].reshape(1, -1),
        "w3": tp["W3"], "b3": tp["b3"].reshape(1, -1),
        "tg": tp["ln_g"].reshape(1, -1), "tb": tp["ln_b"].reshape(1, -1),
        "bbw": params["bb_W"], "bbb": params["bb_b"].reshape(1, -1),
        "wne": ep_["W_node"], "bne": ep_["b_node"].reshape(1, -1),
        "wns": sp_["W_node"], "bns": sp_["b_node"].reshape(1, -1),
    }
    s3, nr9, nt, he, hse = _node_final(
        agg2_p[0], agg2_p[1], s1, maskc, noisec, rot9, trans, nf, c)

    he_p = jnp.pad(he, ((0, N_PAD - N), (0, 128 - C_Z)))
    hse_p = jnp.pad(hse, ((0, N_PAD - N), (0, 128 - C_Z)))
    hs, hd = _gather_pairs(he_p, src_sp, dst_sp)
    pad64 = ((0, 64), (0, 0))
    ef = _edge_trans(hs, hd, z_sp, jnp.pad(ep_["W1"][:C_Z], pad64),
                     jnp.pad(ep_["W1"][C_Z:2 * C_Z], pad64),
                     ep_["W1"][2 * C_Z:], ep_["b1"].reshape(1, -1), ep_["W2"],
                     ep_["b2"].reshape(1, -1), ep_["ln_g"].reshape(1, -1),
                     ep_["ln_b"].reshape(1, -1))
    hs2, hd2 = _gather_pairs(hse_p, src_sq, dst_sq)
    sef = _edge_trans(hs2, hd2, z_sq, jnp.pad(sp_["W1"][:C_Z], pad64),
                      jnp.pad(sp_["W1"][C_Z:2 * C_Z], pad64),
                      sp_["W1"][2 * C_Z:], sp_["b1"].reshape(1, -1),
                      sp_["W2"], sp_["b2"].reshape(1, -1),
                      sp_["ln_g"].reshape(1, -1), sp_["ln_b"].reshape(1, -1))

    kl = jnp.zeros(8, F32)
    return (s3, nr9.reshape(N, 3, 3), nt, ef[:E_SP], sef[:E_SEQ], kl, kl)


# speedup vs baseline: 1.8483x; 1.5468x over previous
import functools

import numpy as np
import jax
import jax.numpy as jnp
from jax import lax
from jax.experimental import pallas as pl
from jax.experimental.pallas import tpu as pltpu
from jax.experimental.pallas import tpu_sc as plsc

N = 10000
N_PAD = 10112
E_SP = 160000
E_SEQ = 20000
EP_SP = 163840
EP_SEQ = 20480
C_S = 128
C_Z = 64
C_H = 16
NW = 32
CHUNK = 128
RPT = N_PAD // 16
F32 = jnp.float32
PH = lax.Precision.HIGHEST


def _dot(a, b):
    return jnp.dot(a, b, preferred_element_type=F32)


def _dotf(a, b):
    return jnp.dot(a, b, preferred_element_type=F32)


def _ln(x, g, b):
    mu = jnp.mean(x, axis=-1, keepdims=True)
    var = jnp.mean((x - mu) * (x - mu), axis=-1, keepdims=True)
    return (x - mu) * lax.rsqrt(var + 1e-5) * g + b


def _consts():
    c = {}
    a16 = np.zeros((16, 128), np.float32)
    b16 = np.zeros((16, 128), np.float32)
    for t in range(16):
        a16[t, t] = 1
        b16[t, 32 + t] = 1
    g12 = np.zeros((12, 128), np.float32)
    for t in range(12):
        g12[t, 16 + t] = 1
    m1 = np.zeros((1, 128), np.float32)
    m1[0, 28] = 1
    c["A16"], c["B16"], c["G12"], c["M1"] = a16, b16, g12, m1
    ps = np.zeros((128, 64), np.float32)
    pd = np.zeros((128, 64), np.float32)
    for i in range(4):
        for j in range(4):
            k = 4 * i + j
            for cc in range(3):
                ps[16 + 3 * i + cc, 4 * k + cc] = 1
                pd[16 + 3 * j + cc, 4 * k + cc] = 1
    ss = np.zeros((64, 16), np.float32)
    for k in range(16):
        for cc in range(4):
            ss[4 * k + cc, k] = 1
    qs = np.zeros((128, 16), np.float32)
    qd = np.zeros((128, 16), np.float32)
    for t in range(16):
        qs[t, t] = 1
        qd[32 + t, t] = 1
    ms = np.zeros((128, 1), np.float32)
    ms[28, 0] = 1
    c["PS"], c["PD"], c["SS"], c["QS"], c["QD"], c["MS"] = ps, pd, ss, qs, qd, ms
    for cp in range(3):
        ga = np.zeros((9, 12), np.float32)
        gb = np.zeros((12, 12), np.float32)
        for i in range(4):
            for cc in range(3):
                ga[3 * cc + cp, 3 * i + cc] = 1
                gb[3 * i + cp, 3 * i + cc] = 1
        c[f"GA{cp}"], c[f"GB{cp}"] = ga, gb
    t3 = np.zeros((3, 12), np.float32)
    for i in range(4):
        for cc in range(3):
            t3[cc, 3 * i + cc] = 1
    c["T3"] = t3
    for j in range(3):
        ra = np.zeros((9, 9), np.float32)
        rb = np.zeros((9, 9), np.float32)
        ta = np.zeros((9, 3), np.float32)
        tb = np.zeros((3, 3), np.float32)
        for i in range(3):
            for k in range(3):
                ra[3 * i + j, 3 * i + k] = 1
                rb[3 * j + k, 3 * i + k] = 1
            ta[3 * i + j, i] = 1
            tb[j, i] = 1
        c[f"RA{j}"], c[f"RB{j}"], c[f"TA{j}"], c[f"TB{j}"] = ra, rb, ta, tb
    h16 = np.zeros((16, 128), np.float32)
    for t in range(16):
        h16[t, t] = 1
    cm = np.zeros((1, 128), np.float32)
    cm[0, 16] = 1
    cn = np.zeros((1, 128), np.float32)
    cn[0, 17] = 1
    c["H16"], c["CM"], c["CN"] = h16, cm, cn
    sh16 = np.zeros((128, 16), np.float32)
    for t in range(16):
        sh16[t, t] = 1
    scm = np.zeros((128, 1), np.float32)
    scm[16, 0] = 1
    scn = np.zeros((128, 1), np.float32)
    scn[17, 0] = 1
    c["SH16"], c["SCM"], c["SCN"] = sh16, scm, scn
    return {k: jnp.asarray(v) for k, v in c.items()}


def _run_rows(body, row_args, w_args, out_trailing, block, rows):
    nb = rows // block
    in_specs = (
        [pl.BlockSpec((block, a.shape[1]), lambda i: (i, 0)) for a in row_args]
        + [pl.BlockSpec(w.shape, lambda i: (0, 0)) for w in w_args]
    )
    out_specs = [pl.BlockSpec((block, t), lambda i: (i, 0)) for t in out_trailing]
    out_shape = [jax.ShapeDtypeStruct((rows, t), F32) for t in out_trailing]
    f = pl.pallas_call(body, grid=(nb,), in_specs=in_specs,
                       out_specs=out_specs, out_shape=out_shape)
    return f(*row_args, *w_args)


def _node_pre(s, rot9, trans, maskc, wpts, w1a, w1b, c):
    def body(s_ref, rot_ref, tr_ref, mk_ref, wp, wa, wb,
             ga0, ga1, ga2, gb0, gb1, gb2, t3, a16, b16, g12, m1,
             tab_ref):
        s_ = s_ref[...]
        pts = _dot(s_, wp[...])
        rot_ = rot_ref[...]
        g = _dot(tr_ref[...], t3[...])
        for ga, gb in ((ga0, gb0), (ga1, gb1), (ga2, gb2)):
            g = g + _dot(rot_, ga[...]) * _dot(pts, gb[...])
        a = _dotf(s_, wa[...])
        b = _dotf(s_, wb[...])
        mk = mk_ref[...]
        tab_ref[...] = (_dot(a, a16[...]) + _dot(b, b16[...])
                        + _dot(g, g12[...]) + _dot(mk, m1[...]))

    return _run_rows(
        body, [s, rot9, trans, maskc],
        [wpts, w1a, w1b, c["GA0"], c["GA1"], c["GA2"], c["GB0"], c["GB1"],
         c["GB2"], c["T3"], c["A16"], c["B16"], c["G12"], c["M1"]],
        [128], 1000, N)[0]


def _edge_ipmp(sa, sb, z, w1c, w1d, bm1, c):
    def body(sa_ref, sb_ref, z_ref, ps, pd, ss, qs, qd, msk, wz, wd, b1,
             h16, cm, cn, m_ref):
        sa_ = sa_ref[...]
        sb_ = sb_ref[...]
        df = _dot(sa_, ps[...]) - _dot(sb_, pd[...])
        d2 = _dot(df * df, ss[...])
        dist = jnp.sqrt(d2 + 1e-8)
        h = (_dot(sa_, qs[...]) + _dot(sb_, qd[...]) + _dotf(z_ref[...], wz[...])
             + _dot(dist, wd[...]) + b1[...])
        h = jnp.maximum(h, 0.0)
        mk = _dot(sa_, msk[...])
        one = mk * 0.0 + 1.0
        m_ref[...] = (_dot(h * mk, h16[...]) + _dot(mk, cm[...])
                      + _dot(one, cn[...]))

    ep = sa.shape[0]
    return _run_rows(
        body, [sa, sb, z],
        [c["PS"], c["PD"], c["SS"], c["QS"], c["QD"], c["MS"], w1c, w1d, bm1,
         c["H16"], c["CM"], c["CN"]],
        [128], 512, ep)[0]


def _node_mid(a0, a1, s0, maskc, rot9, trans,
              wm2, bm2, wout, bout, lg, lb, wpts, w1a, w1b, c):
    def body(a0_ref, a1_ref, s_ref, mk_ref, rot_ref, tr_ref,
             w2, b2, wo, bo, g_ln, b_ln, wp, wa, wb, sh16, scm, scn,
             ga0, ga1, ga2, gb0, gb1, gb2, t3, a16, b16, g12, m1,
             s1_ref, tab_ref):
        agg32 = a0_ref[...] + a1_ref[...]
        hsum = _dot(agg32, sh16[...])
        msum = _dot(agg32, scm[...])
        deg = _dot(agg32, scn[...])
        agg = (_dotf(hsum, w2[...]) + _dot(msum, b2[...]))
        agg = agg / jnp.maximum(deg, 1.0)
        upd = _dotf(agg, wo[...]) + bo[...]
        mk = mk_ref[...]
        s1 = _ln(s_ref[...] + upd * mk, g_ln[...], b_ln[...])
        pts = _dot(s1, wp[...])
        rot_ = rot_ref[...]
        g = _dot(tr_ref[...], t3[...])
        for ga, gb in ((ga0, gb0), (ga1, gb1), (ga2, gb2)):
            g = g + _dot(rot_, ga[...]) * _dot(pts, gb[...])
        a = _dotf(s1, wa[...])
        b = _dotf(s1, wb[...])
        s1_ref[...] = s1
        tab_ref[...] = (_dot(a, a16[...]) + _dot(b, b16[...])
                        + _dot(g, g12[...]) + _dot(mk, m1[...]))

    return _run_rows(
        body, [a0, a1, s0, maskc, rot9, trans],
        [wm2, bm2, wout, bout, lg, lb, wpts, w1a, w1b, c["SH16"], c["SCM"],
         c["SCN"], c["GA0"], c["GA1"], c["GA2"], c["GB0"], c["GB1"],
         c["GB2"], c["T3"], c["A16"], c["B16"], c["G12"], c["M1"]],
        [128, 128], 1000, N)


def _node_final(a0, a1, s1, maskc, noisec, rot9, trans, p, c):
    def body(a0_ref, a1_ref, s_ref, mk_ref, nz_ref, rot_ref,
             tr_ref, wm2_, bm2_, wo, bo, g2, b2l, w1, b1, w2, b2, w3, b3,
             tg, tb_ln, bbw, bbb, wne, bne, wns, bns, sh16, scm, scn,
             ra0, ra1, ra2, rb0, rb1, rb2, ta0, ta1, ta2, tb0, tb1, tb2,
             s3_ref, nr_ref, nt_ref, he_ref, hse_ref):
        agg32 = a0_ref[...] + a1_ref[...]
        hsum = _dot(agg32, sh16[...])
        msum = _dot(agg32, scm[...])
        deg = _dot(agg32, scn[...])
        agg = (_dotf(hsum, wm2_[...]) + _dot(msum, bm2_[...]))
        agg = agg / jnp.maximum(deg, 1.0)
        upd = _dotf(agg, wo[...]) + bo[...]
        mk = mk_ref[...]
        s2 = _ln(s_ref[...] + upd * mk, g2[...], b2l[...])
        t = jnp.maximum(_dotf(s2, w1[...]) + b1[...], 0.0)
        t = jnp.maximum(_dotf(t, w2[...]) + b2[...], 0.0)
        t = _dotf(t, w3[...]) + b3[...]
        s3 = _ln(s2 + t, tg[...], tb_ln[...]) * mk
        nz = nz_ref[...]
        uv = (_dot(s3 * nz, bbw[...]) + bbb[...]) * nz
        v0 = uv[:, 0:1]
        v1 = uv[:, 1:2]
        v2 = uv[:, 2:3]
        inv = lax.rsqrt(1.0 + v0 * v0 + v1 * v1 + v2 * v2)
        w = inv
        x = v0 * inv
        y = v1 * inv
        zz = v2 * inv
        ru = jnp.concatenate([
            1 - 2 * (y * y + zz * zz), 2 * (x * y - zz * w), 2 * (x * zz + y * w),
            2 * (x * y + zz * w), 1 - 2 * (x * x + zz * zz), 2 * (y * zz - x * w),
            2 * (x * zz - y * w), 2 * (y * zz + x * w), 1 - 2 * (x * x + y * y),
        ], axis=1)
        rot_ = rot_ref[...]
        nr = (_dot(rot_, ra0[...]) * _dot(ru, rb0[...])
              + _dot(rot_, ra1[...]) * _dot(ru, rb1[...])
              + _dot(rot_, ra2[...]) * _dot(ru, rb2[...]))
        v3 = uv[:, 3:6]
        nt = tr_ref[...] + (_dot(rot_, ta0[...]) * _dot(v3, tb0[...])
                            + _dot(rot_, ta1[...]) * _dot(v3, tb1[...])
                            + _dot(rot_, ta2[...]) * _dot(v3, tb2[...]))
        s3_ref[...] = s3
        nr_ref[...] = nr
        nt_ref[...] = nt
        he_ref[...] = _dotf(s3, wne[...]) + bne[...]
        hse_ref[...] = _dotf(s3, wns[...]) + bns[...]

    return _run_rows(
        body, [a0, a1, s1, maskc, noisec, rot9, trans],
        [p["wm2"], p["bm2"], p["wo"], p["bo"], p["g2"], p["b2l"], p["w1"],
         p["b1"], p["w2"], p["b2"], p["w3"], p["b3"], p["tg"], p["tb"],
         p["bbw"], p["bbb"], p["wne"], p["bne"], p["wns"], p["bns"],
         c["SH16"], c["SCM"], c["SCN"],
         c["RA0"], c["RA1"], c["RA2"], c["RB0"], c["RB1"], c["RB2"],
         c["TA0"], c["TA1"], c["TA2"], c["TB0"], c["TB1"], c["TB2"]],
        [128, 9, 3, 64, 64], 1000, N)


def _edge_trans(hs, hd, z, w1s, w1d, w1z, b1, w2, b2, lg, lb):
    def body(hs_ref, hd_ref, z_ref, ws, wd, wz, b1_, w2_, b2_, g_, bl_,
             out_ref):
        z_ = z_ref[...]
        e = (_dotf(hs_ref[...], ws[...]) + _dotf(hd_ref[...], wd[...])
             + _dotf(z_, wz[...]) + b1_[...])
        e = jnp.maximum(e, 0.0)
        e = _dotf(e, w2_[...]) + b2_[...] + z_
        out_ref[...] = _ln(e, g_[...], bl_[...])

    ep = hs.shape[0]
    return _run_rows(body, [hs, hd, z], [w1s, w1d, w1z, b1, w2, b2, lg, lb],
                     [64], 512, ep)[0]


def _gather_pairs(tab, ia, ib):
    ep = ia.shape[0]
    epw = ep // NW
    nch = epw // CHUNK
    mesh = plsc.VectorSubcoreMesh(core_axis_name="c", subcore_axis_name="s")

    @functools.partial(
        pl.kernel,
        out_type=(jax.ShapeDtypeStruct((ep, 128), F32),
                  jax.ShapeDtypeStruct((ep, 128), F32)),
        mesh=mesh,
        scratch_types=[
            pltpu.VMEM((CHUNK,), jnp.int32), pltpu.VMEM((CHUNK,), jnp.int32),
            pltpu.VMEM((CHUNK,), jnp.int32), pltpu.VMEM((CHUNK,), jnp.int32),
            pltpu.VMEM((CHUNK, 128), F32), pltpu.VMEM((CHUNK, 128), F32),
            pltpu.VMEM((CHUNK, 128), F32), pltpu.VMEM((CHUNK, 128), F32),
            pltpu.SemaphoreType.DMA, pltpu.SemaphoreType.DMA,
            pltpu.SemaphoreType.DMA, pltpu.SemaphoreType.DMA,
            pltpu.SemaphoreType.DMA, pltpu.SemaphoreType.DMA,
        ])
    def gk(t_h, ia_h, ib_h, oa_h, ob_h, iva0, ivb0, iva1, ivb1,
           ra0, rb0, ra1, rb1, si0, si1, sg0, sg1, sw0, sw1):
        wid = lax.axis_index("s") * 2 + lax.axis_index("c")
        base0 = wid * epw
        iv = ((iva0, ivb0), (iva1, ivb1))
        rr = ((ra0, rb0), (ra1, rb1))
        si = (si0, si1)
        sg = (sg0, sg1)
        sw = (sw0, sw1)
        idx_c = [None, None]
        w_c = [None, None]
        idx_c[0] = (pltpu.async_copy(ia_h.at[pl.ds(base0, CHUNK)], iva0, si0),
                    pltpu.async_copy(ib_h.at[pl.ds(base0, CHUNK)], ivb0, si0))
        for k in range(nch):
            s = k & 1
            if k + 1 < nch:
                ns = 1 - s
                nb = base0 + (k + 1) * CHUNK
                idx_c[ns] = (
                    pltpu.async_copy(ia_h.at[pl.ds(nb, CHUNK)], iv[ns][0],
                                     si[ns]),
                    pltpu.async_copy(ib_h.at[pl.ds(nb, CHUNK)], iv[ns][1],
                                     si[ns]))
            if w_c[s] is not None:
                w_c[s][0].wait()
                w_c[s][1].wait()
            idx_c[s][0].wait()
            idx_c[s][1].wait()
            ga = pltpu.async_copy(t_h.at[iv[s][0]], rr[s][0], sg[s])
            gb = pltpu.async_copy(t_h.at[iv[s][1]], rr[s][1], sg[s])
            ga.wait()
            gb.wait()
            b = base0 + k * CHUNK
            w_c[s] = (pltpu.async_copy(rr[s][0], oa_h.at[pl.ds(b, CHUNK)],
                                       sw[s]),
                      pltpu.async_copy(rr[s][1], ob_h.at[pl.ds(b, CHUNK)],
                                       sw[s]))
        for s in (0, 1):
            if w_c[s] is not None:
                w_c[s][0].wait()
                w_c[s][1].wait()

    return gk(tab, ia, ib)


def _scatter_segsum(m, idx):
    ep = m.shape[0]
    w = m.shape[1]
    epw = ep // NW
    nch = epw // CHUNK
    z128 = jnp.zeros((RPT, w), F32)
    mesh = plsc.VectorSubcoreMesh(core_axis_name="c", subcore_axis_name="s")

    @functools.partial(
        pl.kernel,
        out_type=jax.ShapeDtypeStruct((2 * N_PAD, w), F32),
        mesh=mesh,
        scratch_types=[
            pltpu.VMEM((CHUNK, w), F32), pltpu.VMEM((CHUNK, w), F32),
            pltpu.VMEM((CHUNK,), jnp.int32), pltpu.VMEM((CHUNK,), jnp.int32),
            pltpu.VMEM_SHARED((N_PAD, w), F32),
            pltpu.SemaphoreType.DMA, pltpu.SemaphoreType.DMA,
            pltpu.SemaphoreType.DMA, pltpu.SemaphoreType.DMA,
        ])
    def sk(m_h, idx_h, z128_h, oa_h, mbuf0, mbuf1, iv0, iv1, agg_sh,
           sl0, sl1, sa0, sa1):
        cid = lax.axis_index("c")
        sid = lax.axis_index("s")
        wid = sid * 2 + cid
        base0 = wid * epw
        mb = (mbuf0, mbuf1)
        iv = (iv0, iv1)
        sl = (sl0, sl1)
        sa = (sa0, sa1)
        pltpu.sync_copy(z128_h, agg_sh.at[pl.ds(sid * RPT, RPT)])
        plsc.subcore_barrier()
        l_c = [None, None]
        a_c = [None, None]
        l_c[0] = (pltpu.async_copy(idx_h.at[pl.ds(base0, CHUNK)], iv0, sl0),
                  pltpu.async_copy(m_h.at[pl.ds(base0, CHUNK)], mbuf0, sl0))
        for k in range(nch):
            s = k & 1
            if k + 1 < nch:
                ns = 1 - s
                if a_c[ns] is not None:
                    a_c[ns].wait()
                nb = base0 + (k + 1) * CHUNK
                l_c[ns] = (
                    pltpu.async_copy(idx_h.at[pl.ds(nb, CHUNK)], iv[ns],
                                     sl[ns]),
                    pltpu.async_copy(m_h.at[pl.ds(nb, CHUNK)], mb[ns],
                                     sl[ns]))
            l_c[s][0].wait()
            l_c[s][1].wait()
            a_c[s] = pltpu.async_copy(mb[s], agg_sh.at[iv[s]], sa[s],
                                      add=True)
        for s in (0, 1):
            if a_c[s] is not None:
                a_c[s].wait()
        plsc.subcore_barrier()
        pltpu.sync_copy(agg_sh.at[pl.ds(sid * RPT, RPT)],
                        oa_h.at[pl.ds(cid * N_PAD + sid * RPT, RPT)])

    return sk(m, idx, z128).reshape(2, N_PAD, w)


def _pad_rows(x, rows):
    return jnp.pad(x, ((0, rows - x.shape[0]), (0, 0)))


def _pad_idx(ix, ep, fill):
    return jnp.pad(ix, (0, ep - ix.shape[0]), constant_values=fill)


def _ipmp_tables_weights(p):
    w1 = p["W_m1"]
    return (p["W_pts"], w1[:C_S], w1[C_S:2 * C_S], w1[2 * C_S:2 * C_S + C_Z],
            w1[2 * C_S + C_Z:], p["b_m1"].reshape(1, -1), p["W_m2"],
            p["b_m2"].reshape(1, -1))


def kernel(node_features, rot, trans, edge_features, seq_edge_features,
           params, edge_index, seq_edge_index, x_mask, noising_mask,
           num_graphs):
    c = _consts()
    maskc = (~x_mask).astype(F32).reshape(N, 1)
    noisec = noising_mask.astype(F32).reshape(N, 1)
    rot9 = rot.reshape(N, 9)

    src_sp = _pad_idx(edge_index[0], EP_SP, 0)
    dst_sp = _pad_idx(edge_index[1], EP_SP, 0)
    dst_sp_sc = _pad_idx(edge_index[1], EP_SP, N)
    z_sp = _pad_rows(edge_features, EP_SP)
    src_sq = _pad_idx(seq_edge_index[0], EP_SEQ, 0)
    dst_sq = _pad_idx(seq_edge_index[1], EP_SEQ, 0)
    dst_sq_sc = _pad_idx(seq_edge_index[1], EP_SEQ, N)
    z_sq = _pad_rows(seq_edge_features, EP_SEQ)

    sp = params["spatial"]
    wpts, w1a, w1b, w1c, w1d, bm1, wm2, bm2 = _ipmp_tables_weights(sp)
    tab = _node_pre(node_features, rot9, trans, maskc, wpts, w1a, w1b, c)
    tab = _pad_rows(tab, N_PAD)
    sa, sb = _gather_pairs(tab, src_sp, dst_sp)
    m = _edge_ipmp(sa, sb, z_sp, w1c, w1d, bm1, c)
    agg_p = _scatter_segsum(m, dst_sp_sc)

    sq = params["seq"]
    wpts2, w2a, w2b, w2c, w2d, bm1q, wm2q, bm2q = _ipmp_tables_weights(sq)
    s1, tab2 = _node_mid(
        agg_p[0], agg_p[1], node_features, maskc, rot9,
        trans, wm2, bm2, sp["W_out"], sp["b_out"].reshape(1, -1),
        params["ln_s1_g"].reshape(1, -1), params["ln_s1_b"].reshape(1, -1),
        wpts2, w2a, w2b, c)
    tab2 = _pad_rows(tab2, N_PAD)
    sa2, sb2 = _gather_pairs(tab2, src_sq, dst_sq)
    m2 = _edge_ipmp(sa2, sb2, z_sq, w2c, w2d, bm1q, c)
    agg2_p = _scatter_segsum(m2, dst_sq_sc)

    tp = params["trans"]
    ep_, sp_ = params["edge"], params["seq_edge"]
    nf = {
        "wm2": wm2q, "bm2": bm2q,
        "wo": sq["W_out"], "bo": sq["b_out"].reshape(1, -1),
        "g2": params["ln_s2_g"].reshape(1, -1),
        "b2l": params["ln_s2_b"].reshape(1, -1),
        "w1": tp["W1"], "b1": tp["b1"].reshape(1, -1),
        "w2": tp["W2"], "b2": tp["b2"].reshape(1, -1),
        "w3": tp["W3"], "b3": tp["b3"].reshape(1, -1),
        "tg": tp["ln_g"].reshape(1, -1), "tb": tp["ln_b"].reshape(1, -1),
        "bbw": params["bb_W"], "bbb": params["bb_b"].reshape(1, -1),
        "wne": ep_["W_node"], "bne": ep_["b_node"].reshape(1, -1),
        "wns": sp_["W_node"], "bns": sp_["b_node"].reshape(1, -1),
    }
    s3, nr9, nt, he, hse = _node_final(
        agg2_p[0], agg2_p[1], s1, maskc, noisec, rot9, trans, nf, c)

    he_p = jnp.pad(he, ((0, N_PAD - N), (0, 128 - C_Z)))
    hse_p = jnp.pad(hse, ((0, N_PAD - N), (0, 128 - C_Z)))
    hs, hd = _gather_pairs(he_p, src_sp, dst_sp)
    pad64 = ((0, 64), (0, 0))
    ef = _edge_trans(hs, hd, z_sp, jnp.pad(ep_["W1"][:C_Z], pad64),
                     jnp.pad(ep_["W1"][C_Z:2 * C_Z], pad64),
                     ep_["W1"][2 * C_Z:], ep_["b1"].reshape(1, -1), ep_["W2"],
                     ep_["b2"].reshape(1, -1), ep_["ln_g"].reshape(1, -1),
                     ep_["ln_b"].reshape(1, -1))
    hs2, hd2 = _gather_pairs(hse_p, src_sq, dst_sq)
    sef = _edge_trans(hs2, hd2, z_sq, jnp.pad(sp_["W1"][:C_Z], pad64),
                      jnp.pad(sp_["W1"][C_Z:2 * C_Z], pad64),
                      sp_["W1"][2 * C_Z:], sp_["b1"].reshape(1, -1),
                      sp_["W2"], sp_["b2"].reshape(1, -1),
                      sp_["ln_g"].reshape(1, -1), sp_["ln_b"].reshape(1, -1))

    kl = jnp.zeros(8, F32)
    return (s3, nr9.reshape(N, 3, 3), nt, ef[:E_SP], sef[:E_SEQ], kl, kl)
